# baseline (speedup 1.0000x reference)
import math

import jax
import jax.numpy as jnp
from jax import lax
from jax.experimental import pallas as pl
from jax.experimental.pallas import tpu as pltpu
from jax.experimental.pallas import tpu_sc as plsc

_B, _C, _H, _W = 4, 96, 224, 224
_NIMG = 3
_SLICE = _H * _W
_NPIX = _B * _SLICE
_BIG = _NPIX + 2
_HBLK = 32

_NTILE = 16
_NL = 200960
_CHUNK = _NL // _NTILE
_TPP = _SLICE // _NTILE
_NJ = _TPP // 16
_NJC = _CHUNK // 16
_TASKS_PER_CORE = (_NIMG * _B) // 2
_BIGF = float(_NL + 7)


def _mean_body(x1, x2, x3, o1, o2, o3):
    for x, o in ((x1, o1), (x2, o2), (x3, o3)):
        o[0] = jnp.sum(x[0], axis=0) / float(_C)


def _channel_means(a1, a2, a3):
    in_spec = pl.BlockSpec((1, _C, _HBLK, _W), lambda b, h: (b, 0, h, 0))
    out_spec = pl.BlockSpec((1, _HBLK, _W), lambda b, h: (b, h, 0))
    out_shape = jax.ShapeDtypeStruct((_B, _H, _W), jnp.float32)
    return pl.pallas_call(
        _mean_body,
        grid=(_B, _H // _HBLK),
        in_specs=[in_spec] * 3,
        out_specs=[out_spec] * 3,
        out_shape=[out_shape] * 3,
    )(a1, a2, a3)


def _mask_body(m1, m2, m3, r1, r2, r3, lab):
    b = pl.program_id(0)
    row = lax.broadcasted_iota(jnp.int32, (_H, _W), 0)
    col = lax.broadcasted_iota(jnp.int32, (_H, _W), 1)
    base = b * _SLICE + row * _W + col + 1
    for i, (m, r) in enumerate(((m1, r1), (m2, r2), (m3, r3))):
        x = m[0]
        mn = jnp.min(x)
        mx = jnp.max(x)
        y = (x - mn) / (mx - mn)
        thr = 0.4 * jnp.max(y)
        msk = y >= thr
        r[0] = msk.astype(jnp.float32)
        lab[i, 0] = jnp.where(msk, base, 0)


def _masks_and_labels(m1, m2, m3):
    mspec = pl.BlockSpec((1, _H, _W), lambda b: (b, 0, 0))
    lspec = pl.BlockSpec((_NIMG, 1, _H, _W), lambda b: (0, b, 0, 0))
    return pl.pallas_call(
        _mask_body,
        grid=(_B,),
        in_specs=[mspec] * 3,
        out_specs=[mspec] * 3 + [lspec],
        out_shape=[jax.ShapeDtypeStruct((_B, _H, _W), jnp.float32)] * 3
        + [jax.ShapeDtypeStruct((_NIMG, _B, _H, _W), jnp.int32)],
    )(m1, m2, m3)


def _shift(x, axis, d, fill, fwd):
    pad_shape = list(x.shape)
    pad_shape[axis] = d
    pad = jnp.full(pad_shape, fill, x.dtype)
    sl = [slice(None)] * x.ndim
    if fwd:
        sl[axis] = slice(0, x.shape[axis] - d)
        return jnp.concatenate([pad, x[tuple(sl)]], axis=axis)
    sl[axis] = slice(d, None)
    return jnp.concatenate([x[tuple(sl)], pad], axis=axis)


def _cc_body(lab_in, lab_out):
    lab_out[...] = lab_in[...]
    mask = lab_in[...] > 0
    brk0 = jnp.where(mask, 0, 1).astype(jnp.int32)

    def _cond(c):
        return c > 0

    def _one_sweep(c):
        lab = lab_out[...]
        v = jnp.where(mask, lab, _BIG)
        for axis, nlev in ((3, 8), (2, 8), (1, 2)):
            for fwd in (True, False):
                brk = brk0
                for k in range(nlev):
                    d = 1 << k
                    vs = _shift(v, axis, d, _BIG, fwd)
                    bs = _shift(brk, axis, d, 1, fwd)
                    v = jnp.minimum(v, jnp.where(brk > 0, _BIG, vs))
                    brk = brk | bs
        new = jnp.where(mask, v, 0)
        lab_out[...] = new
        return jnp.any(new != lab).astype(jnp.int32)

    lax.while_loop(_cond, _one_sweep, jnp.int32(1))


def _label_components_pl(labels0):
    return pl.pallas_call(
        _cc_body,
        out_shape=jax.ShapeDtypeStruct(labels0.shape, jnp.int32),
    )(labels0)


def _stats_body(lab_hbm, out_hbm, idx_v, ones_v, zeros_v, zp_v, cnt_v,
                bc_v, bi_v, wv_v, res_v, stat_v,
                counts_sh, results_sh, stats_sh):
    cid = lax.axis_index("c")
    sid = lax.axis_index("s")
    iota = lax.iota(jnp.int32, 16)

    def _fillo(i, c):
        ones_v[pl.ds(i * 16, 16)] = jnp.full((16,), 1.0, jnp.float32)
        return c

    lax.fori_loop(0, _NJ, _fillo, 0, unroll=8)

    def _fillz(i, c):
        zeros_v[pl.ds(i * 16, 16)] = jnp.zeros((16,), jnp.float32)
        return c

    lax.fori_loop(0, _NJC, _fillz, 0, unroll=8)

    def _fillzp(i, c):
        zp_v[pl.ds(i * 16, 16)] = jnp.zeros((16,), jnp.float32)
        return c

    lax.fori_loop(0, _NJ, _fillzp, 0, unroll=8)

    pltpu.sync_copy(zeros_v, counts_sh.at[pl.ds(sid * _CHUNK, _CHUNK)])
    plsc.subcore_barrier()

    for t_local in range(_TASKS_PER_CORE):
        task = cid * _TASKS_PER_CORE + t_local

        base = task * _SLICE + sid * _TPP
        pltpu.sync_copy(lab_hbm.at[pl.ds(base, _TPP)], idx_v)
        pltpu.sync_copy(ones_v, counts_sh.at[idx_v], add=True)
        plsc.subcore_barrier()

        pltpu.sync_copy(counts_sh.at[pl.ds(sid * _CHUNK, _CHUNK)], cnt_v)
        gbase = sid * _CHUNK

        def _scan(j, carry):
            bc, bi = carry
            vv = cnt_v[pl.ds(j * 16, 16)]
            gi = gbase + j * 16 + iota
            vv = jnp.where(gi == 0, -1.0, vv)
            gif = gi.astype(jnp.float32)
            better = (vv > bc) | ((vv == bc) & (gif < bi))
            return (jnp.where(better, vv, bc), jnp.where(better, gif, bi))

        bc, bi = lax.fori_loop(
            0, _NJC, _scan,
            (jnp.full((16,), -2.0, jnp.float32),
             jnp.full((16,), _BIGF, jnp.float32)), unroll=8)
        bc_v[...] = bc
        bi_v[...] = bi
        pltpu.sync_copy(bc_v, results_sh.at[pl.ds(sid * 32, 16)])
        pltpu.sync_copy(bi_v, results_sh.at[pl.ds(sid * 32 + 16, 16)])
        plsc.subcore_barrier()

        pltpu.sync_copy(results_sh, res_v)
        rc = jnp.full((16,), -2.0, jnp.float32)
        ri = jnp.full((16,), _BIGF, jnp.float32)
        for t in range(_NTILE):
            cv = res_v[pl.ds(t * 32, 16)]
            iv = res_v[pl.ds(t * 32 + 16, 16)]
            better = (cv > rc) | ((cv == rc) & (iv < ri))
            rc = jnp.where(better, cv, rc)
            ri = jnp.where(better, iv, ri)
        win_c = rc[0]
        win_f = ri[0]
        for l in range(1, 16):
            c = rc[l]
            i = ri[l]
            take = (c > win_c) | ((c == win_c) & (i < win_f))
            win_c = jnp.where(take, c, win_c)
            win_f = jnp.where(take, i, win_f)
        win_i = win_f.astype(jnp.int32)

        pbase = sid * _TPP

        def _cent(j, carry):
            cc, ch, cw = carry
            lv = idx_v[pl.ds(j * 16, 16)]
            mf = jnp.where(lv == win_i, 1.0, 0.0)
            p = pbase + j * 16 + iota
            hh = lax.div(p, _W)
            ww = p - hh * _W
            return (cc + mf,
                    ch + hh.astype(jnp.float32) * mf,
                    cw + ww.astype(jnp.float32) * mf)

        z16 = jnp.zeros((16,), jnp.float32)
        cc, ch, cw = lax.fori_loop(0, _NJ, _cent, (z16, z16, z16), unroll=4)
        ccs = cc[0]
        chs = ch[0]
        cws = cw[0]
        for l in range(1, 16):
            ccs = ccs + cc[l]
            chs = chs + ch[l]
            cws = cws + cw[l]
        zv = jnp.zeros((16,), jnp.float32)
        vout = jnp.where(
            iota == 0, zv + ccs,
            jnp.where(iota == 1, zv + chs,
                      jnp.where(iota == 2, zv + cws,
                                jnp.where(iota == 3, zv + win_f, zv))))
        wv_v[...] = vout
        pltpu.sync_copy(wv_v, stats_sh.at[pl.ds(sid * 16, 16)])
        pltpu.sync_copy(zp_v, counts_sh.at[idx_v])
        plsc.subcore_barrier()

        @pl.when(sid == 0)
        def _():
            pltpu.sync_copy(stats_sh, stat_v)
            acc = jnp.zeros((16,), jnp.float32)
            for t in range(_NTILE):
                acc = acc + stat_v[pl.ds(t * 16, 16)]
            acc = jnp.where(iota == 3, jnp.zeros((16,), jnp.float32) + win_f, acc)
            wv_v[...] = acc
            pltpu.sync_copy(wv_v, out_hbm.at[pl.ds(task * 16, 16)])


def _stats_call(lab_flat):
    mesh = plsc.VectorSubcoreMesh(core_axis_name="c", subcore_axis_name="s")
    f = pl.kernel(
        _stats_body,
        mesh=mesh,
        out_type=jax.ShapeDtypeStruct((_NIMG * _B * 16,), jnp.float32),
        scratch_types=[
            pltpu.VMEM((_TPP,), jnp.int32),
            pltpu.VMEM((_TPP,), jnp.float32),
            pltpu.VMEM((_CHUNK,), jnp.float32),
            pltpu.VMEM((_TPP,), jnp.float32),
            pltpu.VMEM((_CHUNK,), jnp.float32),
            pltpu.VMEM((16,), jnp.float32),
            pltpu.VMEM((16,), jnp.float32),
            pltpu.VMEM((16,), jnp.float32),
            pltpu.VMEM((_NTILE * 32,), jnp.float32),
            pltpu.VMEM((_NTILE * 16,), jnp.float32),
            pltpu.VMEM_SHARED((_NL,), jnp.float32),
            pltpu.VMEM_SHARED((_NTILE * 32,), jnp.float32),
            pltpu.VMEM_SHARED((_NTILE * 16,), jnp.float32),
        ],
    )
    return f(lab_flat)


def _spherical_w(t1, p1, t2, p2):
    cosd = jnp.sin(t1) * jnp.sin(t2) + jnp.cos(t1) * jnp.cos(t2) * jnp.cos(p1 - p2)
    w = jnp.arccos(cosd) / math.pi
    return jnp.where(jnp.isnan(w), jnp.zeros_like(w), w)


def kernel(input_1, input_2, input_3):
    m1, m2, m3 = _channel_means(input_1, input_2, input_3)
    r1, r2, r3, labels0 = _masks_and_labels(m1, m2, m3)
    labels = _label_components_pl(labels0)
    stats = _stats_call(labels.reshape(_NIMG * _NPIX))
    s = stats.reshape(_NIMG, _B, 16)
    cnt = s[..., 0]
    phis = s[..., 1] / cnt
    thetas = s[..., 2] / cnt
    phi = (0.5 - phis / _H) * math.pi
    theta = (thetas / _W - 0.5) * 2.0 * math.pi
    w1 = _spherical_w(theta[0], phi[0], theta[1], phi[1]).reshape(_B, 1, 1, 1)
    w2 = _spherical_w(theta[1], phi[1], theta[2], phi[2]).reshape(_B, 1, 1, 1)
    return (w1, w2,
            r1.reshape(_B, 1, _H, _W),
            r2.reshape(_B, 1, _H, _W),
            r3.reshape(_B, 1, _H, _W))


# speedup vs baseline: 1.1259x; 1.1259x over previous
import math

import jax
import jax.numpy as jnp
from jax import lax
from jax.experimental import pallas as pl
from jax.experimental.pallas import tpu as pltpu
from jax.experimental.pallas import tpu_sc as plsc

_B, _C, _H, _W = 4, 96, 224, 224
_NIMG = 3
_SLICE = _H * _W
_NPIX = _B * _SLICE
_BIG = _NPIX + 2
_HBLK = 32

_NTILE = 16
_NL = 200960
_CHUNK = _NL // _NTILE
_TPP = _SLICE // _NTILE
_NJ = _TPP // 16
_NJC = _CHUNK // 16
_TASKS_PER_CORE = (_NIMG * _B) // 2
_BIGF = float(_NL + 7)


def _mean_body(x1, x2, x3, o1, o2, o3):
    for x, o in ((x1, o1), (x2, o2), (x3, o3)):
        o[0] = jnp.sum(x[0], axis=0) / float(_C)


def _channel_means(a1, a2, a3):
    in_spec = pl.BlockSpec((1, _C, _HBLK, _W), lambda b, h: (b, 0, h, 0))
    out_spec = pl.BlockSpec((1, _HBLK, _W), lambda b, h: (b, h, 0))
    out_shape = jax.ShapeDtypeStruct((_B, _H, _W), jnp.float32)
    return pl.pallas_call(
        _mean_body,
        grid=(_B, _H // _HBLK),
        in_specs=[in_spec] * 3,
        out_specs=[out_spec] * 3,
        out_shape=[out_shape] * 3,
    )(a1, a2, a3)


def _mask_body(m1, m2, m3, r1, r2, r3, lab):
    b = pl.program_id(0)
    row = lax.broadcasted_iota(jnp.int32, (_H, _W), 0)
    col = lax.broadcasted_iota(jnp.int32, (_H, _W), 1)
    base = b * _SLICE + row * _W + col + 1
    for i, (m, r) in enumerate(((m1, r1), (m2, r2), (m3, r3))):
        x = m[0]
        mn = jnp.min(x)
        mx = jnp.max(x)
        y = (x - mn) / (mx - mn)
        thr = 0.4 * jnp.max(y)
        msk = y >= thr
        r[0] = msk.astype(jnp.float32)
        lab[i, 0] = jnp.where(msk, base, 0)


def _masks_and_labels(m1, m2, m3):
    mspec = pl.BlockSpec((1, _H, _W), lambda b: (b, 0, 0))
    lspec = pl.BlockSpec((_NIMG, 1, _H, _W), lambda b: (0, b, 0, 0))
    return pl.pallas_call(
        _mask_body,
        grid=(_B,),
        in_specs=[mspec] * 3,
        out_specs=[mspec] * 3 + [lspec],
        out_shape=[jax.ShapeDtypeStruct((_B, _H, _W), jnp.float32)] * 3
        + [jax.ShapeDtypeStruct((_NIMG, _B, _H, _W), jnp.int32)],
    )(m1, m2, m3)


def _shift(x, axis, d, fill, fwd):
    pad_shape = list(x.shape)
    pad_shape[axis] = d
    pad = jnp.full(pad_shape, fill, x.dtype)
    sl = [slice(None)] * x.ndim
    if fwd:
        sl[axis] = slice(0, x.shape[axis] - d)
        return jnp.concatenate([pad, x[tuple(sl)]], axis=axis)
    sl[axis] = slice(d, None)
    return jnp.concatenate([x[tuple(sl)], pad], axis=axis)


def _cc_body(lab_in, lab_out):
    lab_out[...] = lab_in[...]
    mask = lab_in[...] > 0
    brk0 = jnp.where(mask, 0, 1).astype(jnp.int32)

    def _cond(c):
        return c > 0

    def _one_sweep(c):
        lab = lab_out[...]
        v = jnp.where(mask, lab, _BIG)
        for axis, nlev in ((3, 8), (2, 8), (1, 2)):
            for fwd in (True, False):
                brk = brk0
                for k in range(nlev):
                    d = 1 << k
                    vs = _shift(v, axis, d, _BIG, fwd)
                    bs = _shift(brk, axis, d, 1, fwd)
                    v = jnp.minimum(v, jnp.where(brk > 0, _BIG, vs))
                    brk = brk | bs
        new = jnp.where(mask, v, 0)
        lab_out[...] = new
        return jnp.any(new != lab).astype(jnp.int32)

    lax.while_loop(_cond, _one_sweep, jnp.int32(1))


def _label_components_pl(labels0):
    return pl.pallas_call(
        _cc_body,
        out_shape=jax.ShapeDtypeStruct(labels0.shape, jnp.int32),
    )(labels0)


def _stats_body(lab_hbm, out_hbm, idx_v, ones_v, zeros_v, cnt_v,
                bc_v, bi_v, wv_v, res_v, stat_v,
                counts_sh, results_sh, stats_sh):
    cid = lax.axis_index("c")
    sid = lax.axis_index("s")
    iota = lax.iota(jnp.int32, 16)

    def _fillo(i, c):
        ones_v[pl.ds(i * 16, 16)] = jnp.full((16,), 1.0, jnp.float32)
        return c

    lax.fori_loop(0, _NJ, _fillo, 0, unroll=8)

    def _fillz(i, c):
        zeros_v[pl.ds(i * 16, 16)] = jnp.zeros((16,), jnp.float32)
        return c

    lax.fori_loop(0, _NJC, _fillz, 0, unroll=8)

    for t_local in range(_TASKS_PER_CORE):
        task = cid * _TASKS_PER_CORE + t_local

        pltpu.sync_copy(zeros_v, counts_sh.at[pl.ds(sid * _CHUNK, _CHUNK)])
        plsc.subcore_barrier()

        base = task * _SLICE + sid * _TPP
        pltpu.sync_copy(lab_hbm.at[pl.ds(base, _TPP)], idx_v)
        pltpu.sync_copy(ones_v, counts_sh.at[idx_v], add=True)
        plsc.subcore_barrier()

        pltpu.sync_copy(counts_sh.at[pl.ds(sid * _CHUNK, _CHUNK)], cnt_v)
        gbase = sid * _CHUNK

        def _scan(j, carry):
            bc, bi = carry
            vv = cnt_v[pl.ds(j * 16, 16)]
            gi = gbase + j * 16 + iota
            vv = jnp.where(gi == 0, -1.0, vv)
            gif = gi.astype(jnp.float32)
            better = (vv > bc) | ((vv == bc) & (gif < bi))
            return (jnp.where(better, vv, bc), jnp.where(better, gif, bi))

        bc, bi = lax.fori_loop(
            0, _NJC, _scan,
            (jnp.full((16,), -2.0, jnp.float32),
             jnp.full((16,), _BIGF, jnp.float32)), unroll=8)
        bc_v[...] = bc
        bi_v[...] = bi
        pltpu.sync_copy(bc_v, results_sh.at[pl.ds(sid * 32, 16)])
        pltpu.sync_copy(bi_v, results_sh.at[pl.ds(sid * 32 + 16, 16)])
        plsc.subcore_barrier()

        pltpu.sync_copy(results_sh, res_v)
        rc = jnp.full((16,), -2.0, jnp.float32)
        ri = jnp.full((16,), _BIGF, jnp.float32)
        for t in range(_NTILE):
            cv = res_v[pl.ds(t * 32, 16)]
            iv = res_v[pl.ds(t * 32 + 16, 16)]
            better = (cv > rc) | ((cv == rc) & (iv < ri))
            rc = jnp.where(better, cv, rc)
            ri = jnp.where(better, iv, ri)
        win_c = rc[0]
        win_f = ri[0]
        for l in range(1, 16):
            c = rc[l]
            i = ri[l]
            take = (c > win_c) | ((c == win_c) & (i < win_f))
            win_c = jnp.where(take, c, win_c)
            win_f = jnp.where(take, i, win_f)
        win_i = win_f.astype(jnp.int32)

        pbase = sid * _TPP

        def _cent(j, carry):
            cc, ch, cw = carry
            lv = idx_v[pl.ds(j * 16, 16)]
            mf = jnp.where(lv == win_i, 1.0, 0.0)
            p = pbase + j * 16 + iota
            hh = lax.div(p, _W)
            ww = p - hh * _W
            return (cc + mf,
                    ch + hh.astype(jnp.float32) * mf,
                    cw + ww.astype(jnp.float32) * mf)

        z16 = jnp.zeros((16,), jnp.float32)
        cc, ch, cw = lax.fori_loop(0, _NJ, _cent, (z16, z16, z16), unroll=4)
        ccs = cc[0]
        chs = ch[0]
        cws = cw[0]
        for l in range(1, 16):
            ccs = ccs + cc[l]
            chs = chs + ch[l]
            cws = cws + cw[l]
        zv = jnp.zeros((16,), jnp.float32)
        vout = jnp.where(
            iota == 0, zv + ccs,
            jnp.where(iota == 1, zv + chs,
                      jnp.where(iota == 2, zv + cws,
                                jnp.where(iota == 3, zv + win_f, zv))))
        wv_v[...] = vout
        pltpu.sync_copy(wv_v, stats_sh.at[pl.ds(sid * 16, 16)])
        plsc.subcore_barrier()

        @pl.when(sid == 0)
        def _():
            pltpu.sync_copy(stats_sh, stat_v)
            acc = jnp.zeros((16,), jnp.float32)
            for t in range(_NTILE):
                acc = acc + stat_v[pl.ds(t * 16, 16)]
            acc = jnp.where(iota == 3, jnp.zeros((16,), jnp.float32) + win_f, acc)
            wv_v[...] = acc
            pltpu.sync_copy(wv_v, out_hbm.at[pl.ds(task * 16, 16)])


def _stats_call(lab_flat):
    mesh = plsc.VectorSubcoreMesh(core_axis_name="c", subcore_axis_name="s")
    f = pl.kernel(
        _stats_body,
        mesh=mesh,
        out_type=jax.ShapeDtypeStruct((_NIMG * _B * 16,), jnp.float32),
        scratch_types=[
            pltpu.VMEM((_TPP,), jnp.int32),
            pltpu.VMEM((_TPP,), jnp.float32),
            pltpu.VMEM((_CHUNK,), jnp.float32),
            pltpu.VMEM((_CHUNK,), jnp.float32),
            pltpu.VMEM((16,), jnp.float32),
            pltpu.VMEM((16,), jnp.float32),
            pltpu.VMEM((16,), jnp.float32),
            pltpu.VMEM((_NTILE * 32,), jnp.float32),
            pltpu.VMEM((_NTILE * 16,), jnp.float32),
            pltpu.VMEM_SHARED((_NL,), jnp.float32),
            pltpu.VMEM_SHARED((_NTILE * 32,), jnp.float32),
            pltpu.VMEM_SHARED((_NTILE * 16,), jnp.float32),
        ],
    )
    return f(lab_flat)


def _spherical_w(t1, p1, t2, p2):
    cosd = jnp.sin(t1) * jnp.sin(t2) + jnp.cos(t1) * jnp.cos(t2) * jnp.cos(p1 - p2)
    w = jnp.arccos(cosd) / math.pi
    return jnp.where(jnp.isnan(w), jnp.zeros_like(w), w)


def kernel(input_1, input_2, input_3):
    m1, m2, m3 = _channel_means(input_1, input_2, input_3)
    r1, r2, r3, labels0 = _masks_and_labels(m1, m2, m3)
    labels = _label_components_pl(labels0)
    stats = _stats_call(labels.reshape(_NIMG * _NPIX))
    s = stats.reshape(_NIMG, _B, 16)
    cnt = s[..., 0]
    phis = s[..., 1] / cnt
    thetas = s[..., 2] / cnt
    phi = (0.5 - phis / _H) * math.pi
    theta = (thetas / _W - 0.5) * 2.0 * math.pi
    w1 = _spherical_w(theta[0], phi[0], theta[1], phi[1]).reshape(_B, 1, 1, 1)
    w2 = _spherical_w(theta[1], phi[1], theta[2], phi[2]).reshape(_B, 1, 1, 1)
    return (w1, w2,
            r1.reshape(_B, 1, _H, _W),
            r2.reshape(_B, 1, _H, _W),
            r3.reshape(_B, 1, _H, _W))


# speedup vs baseline: 1.5003x; 1.3326x over previous
import math

import jax
import jax.numpy as jnp
from jax import lax
from jax.experimental import pallas as pl
from jax.experimental.pallas import tpu as pltpu
from jax.experimental.pallas import tpu_sc as plsc

_B, _C, _H, _W = 4, 96, 224, 224
_NIMG = 3
_SLICE = _H * _W
_NPIX = _B * _SLICE
_BIG = _NPIX + 2
_HBLK = 32

_NTILE = 16
_NL = 200960
_CHUNK = _NL // _NTILE
_TPP = _SLICE // _NTILE
_NJ = _TPP // 16
_NJC = _CHUNK // 16
_TASKS_PER_CORE = (_NIMG * _B) // 2
_BIGF = float(_NL + 7)


def _mean_body(x1, x2, x3, o1, o2, o3):
    for x, o in ((x1, o1), (x2, o2), (x3, o3)):
        o[0] = jnp.sum(x[0], axis=0) / float(_C)


def _channel_means(a1, a2, a3):
    in_spec = pl.BlockSpec((1, _C, _HBLK, _W), lambda b, h: (b, 0, h, 0))
    out_spec = pl.BlockSpec((1, _HBLK, _W), lambda b, h: (b, h, 0))
    out_shape = jax.ShapeDtypeStruct((_B, _H, _W), jnp.float32)
    return pl.pallas_call(
        _mean_body,
        grid=(_B, _H // _HBLK),
        in_specs=[in_spec] * 3,
        out_specs=[out_spec] * 3,
        out_shape=[out_shape] * 3,
    )(a1, a2, a3)


def _mask_body(m1, m2, m3, r1, r2, r3, lab):
    b = pl.program_id(0)
    row = lax.broadcasted_iota(jnp.int32, (_H, _W), 0)
    col = lax.broadcasted_iota(jnp.int32, (_H, _W), 1)
    base = b * _SLICE + row * _W + col + 1
    for i, (m, r) in enumerate(((m1, r1), (m2, r2), (m3, r3))):
        x = m[0]
        mn = jnp.min(x)
        mx = jnp.max(x)
        y = (x - mn) / (mx - mn)
        thr = 0.4 * jnp.max(y)
        msk = y >= thr
        r[0] = msk.astype(jnp.float32)
        lab[i, 0] = jnp.where(msk, base, 0)


def _masks_and_labels(m1, m2, m3):
    mspec = pl.BlockSpec((1, _H, _W), lambda b: (b, 0, 0))
    lspec = pl.BlockSpec((_NIMG, 1, _H, _W), lambda b: (0, b, 0, 0))
    return pl.pallas_call(
        _mask_body,
        grid=(_B,),
        in_specs=[mspec] * 3,
        out_specs=[mspec] * 3 + [lspec],
        out_shape=[jax.ShapeDtypeStruct((_B, _H, _W), jnp.float32)] * 3
        + [jax.ShapeDtypeStruct((_NIMG, _B, _H, _W), jnp.int32)],
    )(m1, m2, m3)


def _shift(x, axis, d, fill, fwd):
    pad_shape = list(x.shape)
    pad_shape[axis] = d
    pad = jnp.full(pad_shape, fill, x.dtype)
    sl = [slice(None)] * x.ndim
    if fwd:
        sl[axis] = slice(0, x.shape[axis] - d)
        return jnp.concatenate([pad, x[tuple(sl)]], axis=axis)
    sl[axis] = slice(d, None)
    return jnp.concatenate([x[tuple(sl)], pad], axis=axis)


def _cc_body(lab_in, lab_out):
    lab_out[...] = lab_in[...]
    mask = lab_in[...] > 0
    brk0 = jnp.where(mask, 0, 1).astype(jnp.int32)
    M = jnp.int32(_BIG + 1)
    INF = jnp.int32(1 << 30)
    axes = ((3, 8), (2, 8), (1, 2))
    mg = {}
    for axis, nlev in axes:
        for fwd in (True, False):
            g = brk0
            for k in range(nlev):
                g = g + _shift(g, axis, 1 << k, 0, fwd)
            mg[(axis, fwd)] = M * g

    def _cond(c):
        return c > 0

    def _one_sweep(c):
        lab = lab_out[...]
        v = jnp.where(mask, lab, _BIG)
        for axis, nlev in axes:
            for fwd in (True, False):
                key = v - mg[(axis, fwd)]
                for k in range(nlev):
                    key = jnp.minimum(key, _shift(key, axis, 1 << k, INF, fwd))
                v = jnp.where(mask, key + mg[(axis, fwd)], _BIG)
        new = jnp.where(mask, v, 0)
        lab_out[...] = new
        return jnp.any(new != lab).astype(jnp.int32)

    lax.while_loop(_cond, _one_sweep, jnp.int32(1))


def _label_components_pl(labels0):
    return pl.pallas_call(
        _cc_body,
        out_shape=jax.ShapeDtypeStruct(labels0.shape, jnp.int32),
    )(labels0)


def _stats_body(lab_hbm, out_hbm, idx_v, ones_v, zeros_v, cnt_v,
                bc_v, bi_v, wv_v, res_v, stat_v,
                counts_sh, results_sh, stats_sh):
    cid = lax.axis_index("c")
    sid = lax.axis_index("s")
    iota = lax.iota(jnp.int32, 16)

    def _fillo(i, c):
        ones_v[pl.ds(i * 16, 16)] = jnp.full((16,), 1.0, jnp.float32)
        return c

    lax.fori_loop(0, _NJ, _fillo, 0, unroll=8)

    def _fillz(i, c):
        zeros_v[pl.ds(i * 16, 16)] = jnp.zeros((16,), jnp.float32)
        return c

    lax.fori_loop(0, _NJC, _fillz, 0, unroll=8)

    for t_local in range(_TASKS_PER_CORE):
        task = cid * _TASKS_PER_CORE + t_local

        pltpu.sync_copy(zeros_v, counts_sh.at[pl.ds(sid * _CHUNK, _CHUNK)])
        plsc.subcore_barrier()

        base = task * _SLICE + sid * _TPP
        pltpu.sync_copy(lab_hbm.at[pl.ds(base, _TPP)], idx_v)
        pltpu.sync_copy(ones_v, counts_sh.at[idx_v], add=True)
        plsc.subcore_barrier()

        pltpu.sync_copy(counts_sh.at[pl.ds(sid * _CHUNK, _CHUNK)], cnt_v)
        gbase = sid * _CHUNK

        def _scan(j, carry):
            bc, bi = carry
            vv = cnt_v[pl.ds(j * 16, 16)]
            gi = gbase + j * 16 + iota
            vv = jnp.where(gi == 0, -1.0, vv)
            gif = gi.astype(jnp.float32)
            better = (vv > bc) | ((vv == bc) & (gif < bi))
            return (jnp.where(better, vv, bc), jnp.where(better, gif, bi))

        bc, bi = lax.fori_loop(
            0, _NJC, _scan,
            (jnp.full((16,), -2.0, jnp.float32),
             jnp.full((16,), _BIGF, jnp.float32)), unroll=8)
        bc_v[...] = bc
        bi_v[...] = bi
        pltpu.sync_copy(bc_v, results_sh.at[pl.ds(sid * 32, 16)])
        pltpu.sync_copy(bi_v, results_sh.at[pl.ds(sid * 32 + 16, 16)])
        plsc.subcore_barrier()

        pltpu.sync_copy(results_sh, res_v)
        rc = jnp.full((16,), -2.0, jnp.float32)
        ri = jnp.full((16,), _BIGF, jnp.float32)
        for t in range(_NTILE):
            cv = res_v[pl.ds(t * 32, 16)]
            iv = res_v[pl.ds(t * 32 + 16, 16)]
            better = (cv > rc) | ((cv == rc) & (iv < ri))
            rc = jnp.where(better, cv, rc)
            ri = jnp.where(better, iv, ri)
        win_c = rc[0]
        win_f = ri[0]
        for l in range(1, 16):
            c = rc[l]
            i = ri[l]
            take = (c > win_c) | ((c == win_c) & (i < win_f))
            win_c = jnp.where(take, c, win_c)
            win_f = jnp.where(take, i, win_f)
        win_i = win_f.astype(jnp.int32)

        pbase = sid * _TPP

        def _cent(j, carry):
            cc, ch, cw = carry
            lv = idx_v[pl.ds(j * 16, 16)]
            mf = jnp.where(lv == win_i, 1.0, 0.0)
            p = pbase + j * 16 + iota
            hh = lax.div(p, _W)
            ww = p - hh * _W
            return (cc + mf,
                    ch + hh.astype(jnp.float32) * mf,
                    cw + ww.astype(jnp.float32) * mf)

        z16 = jnp.zeros((16,), jnp.float32)
        cc, ch, cw = lax.fori_loop(0, _NJ, _cent, (z16, z16, z16), unroll=4)
        ccs = cc[0]
        chs = ch[0]
        cws = cw[0]
        for l in range(1, 16):
            ccs = ccs + cc[l]
            chs = chs + ch[l]
            cws = cws + cw[l]
        zv = jnp.zeros((16,), jnp.float32)
        vout = jnp.where(
            iota == 0, zv + ccs,
            jnp.where(iota == 1, zv + chs,
                      jnp.where(iota == 2, zv + cws,
                                jnp.where(iota == 3, zv + win_f, zv))))
        wv_v[...] = vout
        pltpu.sync_copy(wv_v, stats_sh.at[pl.ds(sid * 16, 16)])
        plsc.subcore_barrier()

        @pl.when(sid == 0)
        def _():
            pltpu.sync_copy(stats_sh, stat_v)
            acc = jnp.zeros((16,), jnp.float32)
            for t in range(_NTILE):
                acc = acc + stat_v[pl.ds(t * 16, 16)]
            acc = jnp.where(iota == 3, jnp.zeros((16,), jnp.float32) + win_f, acc)
            wv_v[...] = acc
            pltpu.sync_copy(wv_v, out_hbm.at[pl.ds(task * 16, 16)])


def _stats_call(lab_flat):
    mesh = plsc.VectorSubcoreMesh(core_axis_name="c", subcore_axis_name="s")
    f = pl.kernel(
        _stats_body,
        mesh=mesh,
        out_type=jax.ShapeDtypeStruct((_NIMG * _B * 16,), jnp.float32),
        scratch_types=[
            pltpu.VMEM((_TPP,), jnp.int32),
            pltpu.VMEM((_TPP,), jnp.float32),
            pltpu.VMEM((_CHUNK,), jnp.float32),
            pltpu.VMEM((_CHUNK,), jnp.float32),
            pltpu.VMEM((16,), jnp.float32),
            pltpu.VMEM((16,), jnp.float32),
            pltpu.VMEM((16,), jnp.float32),
            pltpu.VMEM((_NTILE * 32,), jnp.float32),
            pltpu.VMEM((_NTILE * 16,), jnp.float32),
            pltpu.VMEM_SHARED((_NL,), jnp.float32),
            pltpu.VMEM_SHARED((_NTILE * 32,), jnp.float32),
            pltpu.VMEM_SHARED((_NTILE * 16,), jnp.float32),
        ],
    )
    return f(lab_flat)


def _spherical_w(t1, p1, t2, p2):
    cosd = jnp.sin(t1) * jnp.sin(t2) + jnp.cos(t1) * jnp.cos(t2) * jnp.cos(p1 - p2)
    w = jnp.arccos(cosd) / math.pi
    return jnp.where(jnp.isnan(w), jnp.zeros_like(w), w)


def kernel(input_1, input_2, input_3):
    m1, m2, m3 = _channel_means(input_1, input_2, input_3)
    r1, r2, r3, labels0 = _masks_and_labels(m1, m2, m3)
    labels = _label_components_pl(labels0)
    stats = _stats_call(labels.reshape(_NIMG * _NPIX))
    s = stats.reshape(_NIMG, _B, 16)
    cnt = s[..., 0]
    phis = s[..., 1] / cnt
    thetas = s[..., 2] / cnt
    phi = (0.5 - phis / _H) * math.pi
    theta = (thetas / _W - 0.5) * 2.0 * math.pi
    w1 = _spherical_w(theta[0], phi[0], theta[1], phi[1]).reshape(_B, 1, 1, 1)
    w2 = _spherical_w(theta[1], phi[1], theta[2], phi[2]).reshape(_B, 1, 1, 1)
    return (w1, w2,
            r1.reshape(_B, 1, _H, _W),
            r2.reshape(_B, 1, _H, _W),
            r3.reshape(_B, 1, _H, _W))
